# baseline (speedup 1.0000x reference)
import functools

import jax
import jax.numpy as jnp
from jax.experimental import pallas as pl
import jax.experimental.pallas.tpu as pltpu

MEM = 512
DIM = 512
BS = 8
HW = 1024
N = BS * HW
PJ = 1
PB = HW // PJ
NSTEP = BS * PJ
QSTEP = 8
QB = N // QSTEP
INV_T = 10.0


def _fused_kernel(q_ref, k_ref,
                  uq_ref, sm_ref, sq_ref, um_ref, sep_ref, comp_ref,
                  mn_ref, pk_ref, e_ref, colsum_ref, colmax_ref,
                  rawnum_ref, rawden_ref):
    s = pl.program_id(0)
    first = s == 0

    @pl.when(first)
    def _():
        kb = k_ref[...]
        kn2 = jnp.sum(kb * kb, axis=1)
        kinv = 1.0 / jnp.maximum(jnp.sqrt(kn2), 1e-12)
        mn_ref[...] = kb * kinv[:, None]
        ksum = jnp.sum(kb, axis=1)
        lane = jax.lax.broadcasted_iota(jnp.int32, (MEM, 128), 1)
        pk_ref[...] = jnp.where(
            lane == 0, ksum[:, None],
            jnp.where(lane == 1, kn2[:, None], 0.0))

    @pl.when(s < NSTEP)
    def _():
        mn = mn_ref[...]
        qb = q_ref[0]
        qnorm2 = jnp.sum(qb * qb, axis=0)
        qinv = 1.0 / jnp.maximum(jnp.sqrt(qnorm2), 1e-12)
        qn = qb * qinv[None, :]
        qn2 = qnorm2 * qinv * qinv

        score = jax.lax.dot_general(
            qn, mn, (((0,), (1,)), ((), ())),
            preferred_element_type=jnp.float32)

        e = jnp.exp(score * INV_T)
        rowsum = jnp.sum(e, axis=1)
        sm = e * (1.0 / rowsum)[:, None]
        sm_ref[...] = sm
        e_ref[pl.ds(s * PB, PB), :] = e

        csum = jnp.sum(e, axis=0)[None, :]
        cmax = jnp.max(e, axis=0)[None, :]

        emax = jnp.max(e, axis=1)
        b0 = e == emax[:, None]
        w0 = jnp.where(b0, emax[:, None], 0.0)
        masked = jnp.where(b0, 0.0, e)
        e1 = jnp.max(masked, axis=1)
        f1 = jnp.where(masked == e1[:, None], 1.0, 0.0)

        pk = pk_ref[...]
        einv = 1.0 / emax
        gat0 = jnp.dot(w0, pk, preferred_element_type=jnp.float32)
        gat1 = jnp.dot(f1, pk, preferred_element_type=jnp.float32)
        ks0 = gat0[:, 0] * einv
        kn20 = gat0[:, 1] * einv
        ks1 = gat1[:, 0]
        kn21 = gat1[:, 1]
        s0 = jnp.log(emax) * (1.0 / INV_T)
        s1 = jnp.log(e1) * (1.0 / INV_T)

        rsq = jnp.sum(qn, axis=0)

        base0 = qn2 + kn20 - 2.0 * jnp.sqrt(kn20) * s0
        base1 = qn2 + kn21 - 2.0 * jnp.sqrt(kn21) * s1
        eps_d = DIM * 1e-12
        dp = jnp.sqrt(jnp.maximum(base0 + 2e-6 * (rsq - ks0) + eps_d, 0.0))
        dn = jnp.sqrt(jnp.maximum(base1 + 2e-6 * (rsq - ks1) + eps_d, 0.0))
        sep = jnp.sum(jnp.maximum(dp - dn + 1.0, 0.0)).reshape(1, 1)
        comp = jnp.sum(base0).reshape(1, 1)

        part = jax.lax.dot_general(
            w0, qn, (((0,), (1,)), ((), ())),
            preferred_element_type=jnp.float32)
        dpart = jnp.sum(w0, axis=0)[None, :]

        @pl.when(first)
        def _():
            colsum_ref[...] = csum
            colmax_ref[...] = cmax
            rawnum_ref[...] = part
            rawden_ref[...] = dpart
            sep_ref[...] = sep
            comp_ref[...] = comp

        @pl.when(jnp.logical_not(first))
        def _():
            colsum_ref[...] += csum
            colmax_ref[...] = jnp.maximum(colmax_ref[...], cmax)
            rawnum_ref[...] += part
            rawden_ref[...] += dpart
            sep_ref[...] += sep
            comp_ref[...] += comp

        cmT = jax.lax.dot_general(
            k_ref[...], sm, (((0,), (1,)), ((), ())),
            preferred_element_type=jnp.float32)
        uq_ref[0, :DIM, :] = qn
        uq_ref[0, DIM:, :] = cmT

    @pl.when(s >= NSTEP)
    def _():
        i = s - NSTEP
        ec = e_ref[pl.ds(i * QB, QB), :]
        sq_ref[...] = ec * (1.0 / colsum_ref[0, :])[None, :]

        @pl.when(s == NSTEP)
        def _():
            den = rawden_ref[0, :] + 1e-8 * (
                colmax_ref[0, :] + 1e-8 * colsum_ref[0, :])
            qu = rawnum_ref[...] * (1.0 / den)[:, None]
            upd = qu + k_ref[...]
            unorm = jnp.sqrt(jnp.sum(upd * upd, axis=1))
            um_ref[...] = upd * (1.0 / jnp.maximum(unorm, 1e-12))[:, None]


@functools.partial(jax.jit, static_argnames=("interpret",))
def _run(query, keys, interpret=False):
    query_r = query.reshape(BS, DIM, HW)

    fused = pl.pallas_call(
        _fused_kernel,
        grid=(NSTEP + QSTEP,),
        in_specs=[
            pl.BlockSpec(
                (1, DIM, PB),
                lambda s: (jnp.minimum(s, NSTEP - 1) // PJ, 0,
                           jnp.minimum(s, NSTEP - 1) % PJ)),
            pl.BlockSpec((MEM, DIM), lambda s: (0, 0)),
        ],
        out_specs=[
            pl.BlockSpec(
                (1, 2 * DIM, PB),
                lambda s: (jnp.minimum(s, NSTEP - 1) // PJ, 0,
                           jnp.minimum(s, NSTEP - 1) % PJ)),
            pl.BlockSpec((PB, MEM), lambda s: (jnp.minimum(s, NSTEP - 1), 0)),
            pl.BlockSpec((QB, MEM), lambda s: (jnp.maximum(s - NSTEP, 0), 0)),
            pl.BlockSpec((MEM, DIM), lambda s: (0, 0)),
            pl.BlockSpec((1, 1), lambda s: (0, 0)),
            pl.BlockSpec((1, 1), lambda s: (0, 0)),
        ],
        out_shape=[
            jax.ShapeDtypeStruct((BS, 2 * DIM, HW), jnp.float32),
            jax.ShapeDtypeStruct((N, MEM), jnp.float32),
            jax.ShapeDtypeStruct((N, MEM), jnp.float32),
            jax.ShapeDtypeStruct((MEM, DIM), jnp.float32),
            jax.ShapeDtypeStruct((1, 1), jnp.float32),
            jax.ShapeDtypeStruct((1, 1), jnp.float32),
        ],
        scratch_shapes=[
            pltpu.VMEM((MEM, DIM), jnp.float32),
            pltpu.VMEM((MEM, 128), jnp.float32),
            pltpu.VMEM((N, MEM), jnp.float32),
            pltpu.VMEM((1, MEM), jnp.float32),
            pltpu.VMEM((1, MEM), jnp.float32),
            pltpu.VMEM((MEM, DIM), jnp.float32),
            pltpu.VMEM((1, MEM), jnp.float32),
        ],
        interpret=interpret,
    )
    uq_r, sm, sq, um, sep, comp = fused(query_r, keys)

    uq = uq_r.reshape(BS, 2 * DIM, 32, 32)
    sep_s = (sep[0, 0] / N).astype(jnp.float32)
    comp_s = (comp[0, 0] / (N * DIM)).astype(jnp.float32)
    return uq, um, sq, sm, sep_s, comp_s


def kernel(query, keys):
    return _run(query, keys)


# speedup vs baseline: 1.0089x; 1.0089x over previous
import functools

import jax
import jax.numpy as jnp
from jax.experimental import pallas as pl
import jax.experimental.pallas.tpu as pltpu

MEM = 512
DIM = 512
BS = 8
HW = 1024
N = BS * HW
PJ = 1
PB = HW // PJ
NSTEP = BS * PJ
QSTEP = 2
QB = N // QSTEP
INV_T = 10.0


def _fused_kernel(q_ref, k_ref,
                  uq_ref, sm_ref, sq_ref, um_ref, sep_ref, comp_ref,
                  mn_ref, pk_ref, e_ref, colsum_ref, colmax_ref,
                  rawnum_ref, rawden_ref):
    s = pl.program_id(0)
    first = s == 0

    @pl.when(first)
    def _():
        kb = k_ref[...]
        kn2 = jnp.sum(kb * kb, axis=1)
        kinv = 1.0 / jnp.maximum(jnp.sqrt(kn2), 1e-12)
        mn_ref[...] = kb * kinv[:, None]
        ksum = jnp.sum(kb, axis=1)
        lane = jax.lax.broadcasted_iota(jnp.int32, (MEM, 128), 1)
        pk_ref[...] = jnp.where(
            lane == 0, ksum[:, None],
            jnp.where(lane == 1, kn2[:, None], 0.0))

    @pl.when(s < NSTEP)
    def _():
        mn = mn_ref[...]
        qb = q_ref[0]
        qnorm2 = jnp.sum(qb * qb, axis=0)
        qinv = 1.0 / jnp.maximum(jnp.sqrt(qnorm2), 1e-12)
        qn = qb * qinv[None, :]
        qn2 = qnorm2 * qinv * qinv

        score = jax.lax.dot_general(
            qn, mn, (((0,), (1,)), ((), ())),
            preferred_element_type=jnp.float32)

        e = jnp.exp(score * INV_T)
        rowsum = jnp.sum(e, axis=1)
        sm = e * (1.0 / rowsum)[:, None]
        sm_ref[...] = sm
        e_ref[pl.ds(s * PB, PB), :] = e

        csum = jnp.sum(e, axis=0)[None, :]
        cmax = jnp.max(e, axis=0)[None, :]

        emax = jnp.max(e, axis=1)
        b0 = e == emax[:, None]
        w0 = jnp.where(b0, emax[:, None], 0.0)
        masked = jnp.where(b0, 0.0, e)
        e1 = jnp.max(masked, axis=1)
        f1 = jnp.where(masked == e1[:, None], 1.0, 0.0)

        pk = pk_ref[...]
        einv = 1.0 / emax
        gat0 = jnp.dot(w0, pk, preferred_element_type=jnp.float32)
        gat1 = jnp.dot(f1, pk, preferred_element_type=jnp.float32)
        ks0 = gat0[:, 0] * einv
        kn20 = gat0[:, 1] * einv
        ks1 = gat1[:, 0]
        kn21 = gat1[:, 1]
        s0 = jnp.log(emax) * (1.0 / INV_T)
        s1 = jnp.log(e1) * (1.0 / INV_T)

        rsq = jnp.sum(qn, axis=0)

        base0 = qn2 + kn20 - 2.0 * jnp.sqrt(kn20) * s0
        base1 = qn2 + kn21 - 2.0 * jnp.sqrt(kn21) * s1
        eps_d = DIM * 1e-12
        dp = jnp.sqrt(jnp.maximum(base0 + 2e-6 * (rsq - ks0) + eps_d, 0.0))
        dn = jnp.sqrt(jnp.maximum(base1 + 2e-6 * (rsq - ks1) + eps_d, 0.0))
        sep = jnp.sum(jnp.maximum(dp - dn + 1.0, 0.0)).reshape(1, 1)
        comp = jnp.sum(base0).reshape(1, 1)

        part = jax.lax.dot_general(
            w0, qn, (((0,), (1,)), ((), ())),
            preferred_element_type=jnp.float32)
        dpart = jnp.sum(w0, axis=0)[None, :]

        @pl.when(first)
        def _():
            colsum_ref[...] = csum
            colmax_ref[...] = cmax
            rawnum_ref[...] = part
            rawden_ref[...] = dpart
            sep_ref[...] = sep
            comp_ref[...] = comp

        @pl.when(jnp.logical_not(first))
        def _():
            colsum_ref[...] += csum
            colmax_ref[...] = jnp.maximum(colmax_ref[...], cmax)
            rawnum_ref[...] += part
            rawden_ref[...] += dpart
            sep_ref[...] += sep
            comp_ref[...] += comp

        cmT = jax.lax.dot_general(
            k_ref[...], sm, (((0,), (1,)), ((), ())),
            preferred_element_type=jnp.float32)
        uq_ref[0, :DIM, :] = qn
        uq_ref[0, DIM:, :] = cmT

    @pl.when(s >= NSTEP)
    def _():
        i = s - NSTEP
        ec = e_ref[pl.ds(i * QB, QB), :]
        sq_ref[...] = ec * (1.0 / colsum_ref[0, :])[None, :]

        @pl.when(s == NSTEP)
        def _():
            den = rawden_ref[0, :] + 1e-8 * (
                colmax_ref[0, :] + 1e-8 * colsum_ref[0, :])
            qu = rawnum_ref[...] * (1.0 / den)[:, None]
            upd = qu + k_ref[...]
            unorm = jnp.sqrt(jnp.sum(upd * upd, axis=1))
            um_ref[...] = upd * (1.0 / jnp.maximum(unorm, 1e-12))[:, None]


@functools.partial(jax.jit, static_argnames=("interpret",))
def _run(query, keys, interpret=False):
    query_r = query.reshape(BS, DIM, HW)

    fused = pl.pallas_call(
        _fused_kernel,
        grid=(NSTEP + QSTEP,),
        in_specs=[
            pl.BlockSpec(
                (1, DIM, PB),
                lambda s: (jnp.minimum(s, NSTEP - 1) // PJ, 0,
                           jnp.minimum(s, NSTEP - 1) % PJ)),
            pl.BlockSpec((MEM, DIM), lambda s: (0, 0)),
        ],
        out_specs=[
            pl.BlockSpec(
                (1, 2 * DIM, PB),
                lambda s: (jnp.minimum(s, NSTEP - 1) // PJ, 0,
                           jnp.minimum(s, NSTEP - 1) % PJ)),
            pl.BlockSpec((PB, MEM), lambda s: (jnp.minimum(s, NSTEP - 1), 0)),
            pl.BlockSpec((QB, MEM), lambda s: (jnp.maximum(s - NSTEP, 0), 0)),
            pl.BlockSpec((MEM, DIM), lambda s: (0, 0)),
            pl.BlockSpec((1, 1), lambda s: (0, 0)),
            pl.BlockSpec((1, 1), lambda s: (0, 0)),
        ],
        out_shape=[
            jax.ShapeDtypeStruct((BS, 2 * DIM, HW), jnp.float32),
            jax.ShapeDtypeStruct((N, MEM), jnp.float32),
            jax.ShapeDtypeStruct((N, MEM), jnp.float32),
            jax.ShapeDtypeStruct((MEM, DIM), jnp.float32),
            jax.ShapeDtypeStruct((1, 1), jnp.float32),
            jax.ShapeDtypeStruct((1, 1), jnp.float32),
        ],
        scratch_shapes=[
            pltpu.VMEM((MEM, DIM), jnp.float32),
            pltpu.VMEM((MEM, 128), jnp.float32),
            pltpu.VMEM((N, MEM), jnp.float32),
            pltpu.VMEM((1, MEM), jnp.float32),
            pltpu.VMEM((1, MEM), jnp.float32),
            pltpu.VMEM((MEM, DIM), jnp.float32),
            pltpu.VMEM((1, MEM), jnp.float32),
        ],
        compiler_params=pltpu.CompilerParams(
            vmem_limit_bytes=100 * 1024 * 1024),
        interpret=interpret,
    )
    uq_r, sm, sq, um, sep, comp = fused(query_r, keys)

    uq = uq_r.reshape(BS, 2 * DIM, 32, 32)
    sep_s = (sep[0, 0] / N).astype(jnp.float32)
    comp_s = (comp[0, 0] / (N * DIM)).astype(jnp.float32)
    return uq, um, sq, sm, sep_s, comp_s


def kernel(query, keys):
    return _run(query, keys)


# speedup vs baseline: 1.0356x; 1.0265x over previous
import functools

import jax
import jax.numpy as jnp
from jax.experimental import pallas as pl
import jax.experimental.pallas.tpu as pltpu

MEM = 512
DIM = 512
BS = 8
HW = 1024
N = BS * HW
BB = 2
NSTEP = BS // BB
QSTEP = 4
QB = N // QSTEP
INV_T = 10.0


def _fused_kernel(q_ref, k_ref,
                  uq_ref, sm_ref, sq_ref, um_ref, sep_ref, comp_ref,
                  mn_ref, pk_ref, e_ref, colsum_ref, colmax_ref,
                  rawnum_ref, rawden_ref):
    s = pl.program_id(0)
    first = s == 0

    @pl.when(first)
    def _():
        kb = k_ref[...]
        kn2 = jnp.sum(kb * kb, axis=1)
        kinv = 1.0 / jnp.maximum(jnp.sqrt(kn2), 1e-12)
        mn_ref[...] = kb * kinv[:, None]
        ksum = jnp.sum(kb, axis=1)
        lane = jax.lax.broadcasted_iota(jnp.int32, (MEM, 128), 1)
        pk_ref[...] = jnp.where(
            lane == 0, ksum[:, None],
            jnp.where(lane == 1, kn2[:, None], 0.0))

    @pl.when(s < NSTEP)
    def _():
        mn = mn_ref[...]
        pk = pk_ref[...]
        acc = []
        for t in range(BB):
            qb = q_ref[t]
            qnorm2 = jnp.sum(qb * qb, axis=0)
            qinv = 1.0 / jnp.maximum(jnp.sqrt(qnorm2), 1e-12)
            qn = qb * qinv[None, :]
            qn2 = qnorm2 * qinv * qinv

            score = jax.lax.dot_general(
                qn, mn, (((0,), (1,)), ((), ())),
                preferred_element_type=jnp.float32)

            e = jnp.exp(score * INV_T)
            rowsum = jnp.sum(e, axis=1)
            sm = e * (1.0 / rowsum)[:, None]
            sm_ref[pl.ds(t * HW, HW), :] = sm
            e_ref[pl.ds((s * BB + t) * HW, HW), :] = e.astype(jnp.bfloat16)

            csum = jnp.sum(e, axis=0)[None, :]
            cmax = jnp.max(e, axis=0)[None, :]

            emax = jnp.max(e, axis=1)
            b0 = e == emax[:, None]
            w0 = jnp.where(b0, emax[:, None], 0.0)
            masked = jnp.where(b0, 0.0, e)
            e1 = jnp.max(masked, axis=1)
            f1 = jnp.where(masked == e1[:, None], 1.0, 0.0)

            einv = 1.0 / emax
            gat0 = jnp.dot(w0, pk, preferred_element_type=jnp.float32)
            gat1 = jnp.dot(f1, pk, preferred_element_type=jnp.float32)
            ks0 = gat0[:, 0] * einv
            kn20 = gat0[:, 1] * einv
            ks1 = gat1[:, 0]
            kn21 = gat1[:, 1]
            s0 = jnp.log(emax) * (1.0 / INV_T)
            s1 = jnp.log(e1) * (1.0 / INV_T)

            rsq = jnp.sum(qn, axis=0)

            base0 = qn2 + kn20 - 2.0 * jnp.sqrt(kn20) * s0
            base1 = qn2 + kn21 - 2.0 * jnp.sqrt(kn21) * s1
            eps_d = DIM * 1e-12
            dp = jnp.sqrt(jnp.maximum(
                base0 + 2e-6 * (rsq - ks0) + eps_d, 0.0))
            dn = jnp.sqrt(jnp.maximum(
                base1 + 2e-6 * (rsq - ks1) + eps_d, 0.0))
            sep = jnp.sum(jnp.maximum(dp - dn + 1.0, 0.0))
            comp = jnp.sum(base0)

            part = jax.lax.dot_general(
                w0, qn, (((0,), (1,)), ((), ())),
                preferred_element_type=jnp.float32)
            dpart = jnp.sum(w0, axis=0)[None, :]

            cmT = jax.lax.dot_general(
                k_ref[...], sm, (((0,), (1,)), ((), ())),
                preferred_element_type=jnp.float32)
            uq_ref[t, :DIM, :] = qn
            uq_ref[t, DIM:, :] = cmT

            acc.append((csum, cmax, part, dpart, sep, comp))

        csum = acc[0][0] + acc[1][0]
        cmax = jnp.maximum(acc[0][1], acc[1][1])
        part = acc[0][2] + acc[1][2]
        dpart = acc[0][3] + acc[1][3]
        sep = (acc[0][4] + acc[1][4]).reshape(1, 1)
        comp = (acc[0][5] + acc[1][5]).reshape(1, 1)

        @pl.when(first)
        def _():
            colsum_ref[...] = csum
            colmax_ref[...] = cmax
            rawnum_ref[...] = part
            rawden_ref[...] = dpart
            sep_ref[...] = sep
            comp_ref[...] = comp

        @pl.when(jnp.logical_not(first))
        def _():
            colsum_ref[...] += csum
            colmax_ref[...] = jnp.maximum(colmax_ref[...], cmax)
            rawnum_ref[...] += part
            rawden_ref[...] += dpart
            sep_ref[...] += sep
            comp_ref[...] += comp

    @pl.when(s >= NSTEP)
    def _():
        i = s - NSTEP
        ec = e_ref[pl.ds(i * QB, QB), :].astype(jnp.float32)
        sq_ref[...] = ec * (1.0 / colsum_ref[0, :])[None, :]

        @pl.when(s == NSTEP)
        def _():
            den = rawden_ref[0, :] + 1e-8 * (
                colmax_ref[0, :] + 1e-8 * colsum_ref[0, :])
            qu = rawnum_ref[...] * (1.0 / den)[:, None]
            upd = qu + k_ref[...]
            unorm = jnp.sqrt(jnp.sum(upd * upd, axis=1))
            um_ref[...] = upd * (1.0 / jnp.maximum(unorm, 1e-12))[:, None]


@functools.partial(jax.jit, static_argnames=("interpret",))
def _run(query, keys, interpret=False):
    query_r = query.reshape(BS, DIM, HW)

    fused = pl.pallas_call(
        _fused_kernel,
        grid=(NSTEP + QSTEP,),
        in_specs=[
            pl.BlockSpec(
                (BB, DIM, HW),
                lambda s: (jnp.minimum(s, NSTEP - 1), 0, 0)),
            pl.BlockSpec((MEM, DIM), lambda s: (0, 0)),
        ],
        out_specs=[
            pl.BlockSpec(
                (BB, 2 * DIM, HW),
                lambda s: (jnp.minimum(s, NSTEP - 1), 0, 0)),
            pl.BlockSpec((BB * HW, MEM),
                         lambda s: (jnp.minimum(s, NSTEP - 1), 0)),
            pl.BlockSpec((QB, MEM), lambda s: (jnp.maximum(s - NSTEP, 0), 0)),
            pl.BlockSpec((MEM, DIM), lambda s: (0, 0)),
            pl.BlockSpec((1, 1), lambda s: (0, 0)),
            pl.BlockSpec((1, 1), lambda s: (0, 0)),
        ],
        out_shape=[
            jax.ShapeDtypeStruct((BS, 2 * DIM, HW), jnp.float32),
            jax.ShapeDtypeStruct((N, MEM), jnp.float32),
            jax.ShapeDtypeStruct((N, MEM), jnp.float32),
            jax.ShapeDtypeStruct((MEM, DIM), jnp.float32),
            jax.ShapeDtypeStruct((1, 1), jnp.float32),
            jax.ShapeDtypeStruct((1, 1), jnp.float32),
        ],
        scratch_shapes=[
            pltpu.VMEM((MEM, DIM), jnp.float32),
            pltpu.VMEM((MEM, 128), jnp.float32),
            pltpu.VMEM((N, MEM), jnp.bfloat16),
            pltpu.VMEM((1, MEM), jnp.float32),
            pltpu.VMEM((1, MEM), jnp.float32),
            pltpu.VMEM((MEM, DIM), jnp.float32),
            pltpu.VMEM((1, MEM), jnp.float32),
        ],
        compiler_params=pltpu.CompilerParams(
            vmem_limit_bytes=110 * 1024 * 1024),
        interpret=interpret,
    )
    uq_r, sm, sq, um, sep, comp = fused(query_r, keys)

    uq = uq_r.reshape(BS, 2 * DIM, 32, 32)
    sep_s = (sep[0, 0] / N).astype(jnp.float32)
    comp_s = (comp[0, 0] / (N * DIM)).astype(jnp.float32)
    return uq, um, sq, sm, sep_s, comp_s


def kernel(query, keys):
    return _run(query, keys)
